# baseline (speedup 1.0000x reference)
import functools

import jax
import jax.numpy as jnp
from jax import lax
from jax.experimental import pallas as pl
from jax.experimental.pallas import tpu as pltpu
from jax.experimental.pallas import tpu_sc as plsc

_K = 8192
_D = 32
_B = 65536
_BETA = 0.25
_BM = 4096
_KC = 1024
_NJ = _K // _KC
_IDX_CHUNK = 128


def _tc_body(zt_ref, cb2_ref, cn_ref, idx_ref, commit_ref, bd_ref, bm_ref):
    j = pl.program_id(1)
    zt = zt_ref[...]
    cb2 = cb2_ref[pl.ds(j * _KC, _KC), :]
    prod = lax.dot_general(cb2, zt, (((1,), (0,)), ((), ())),
                           preferred_element_type=jnp.float32)
    dists = prod + cn_ref[pl.ds(j * _KC, _KC), :]
    d = jnp.min(dists, axis=0, keepdims=True)
    m = (jnp.argmin(dists, axis=0).astype(jnp.int32)
         .reshape(1, _BM) + j * _KC)

    @pl.when(j == 0)
    def _():
        bd_ref[...] = d
        bm_ref[...] = m

    @pl.when(j != 0)
    def _():
        prev_d = bd_ref[...]
        take = d < prev_d
        bd_ref[...] = jnp.where(take, d, prev_d)
        bm_ref[...] = jnp.where(take, m, bm_ref[...])

    @pl.when((pl.program_id(0) == 0) & (j == 0))
    def _():
        commit_ref[0, 0] = 0.0

    @pl.when(j == _NJ - 1)
    def _():
        idx_ref[...] = bm_ref[...]
        z = zt_ref[...]
        scale = _BETA / (_B * _D)
        commit_ref[0, 0] += (jnp.sum(bd_ref[...]) + jnp.sum(z * z)) * scale


def _tc_search(z_e, codebook, interpret=False):
    call = pl.pallas_call(
        _tc_body,
        interpret=interpret,
        grid=(_B // _BM, _NJ),
        in_specs=[
            pl.BlockSpec((_D, _BM), lambda i, j: (0, i)),
            pl.BlockSpec((_K, _D), lambda i, j: (0, 0)),
            pl.BlockSpec((_K, 1), lambda i, j: (0, 0)),
        ],
        out_specs=[
            pl.BlockSpec((1, _BM), lambda i, j: (0, i)),
            pl.BlockSpec(memory_space=pltpu.SMEM),
        ],
        out_shape=[
            jax.ShapeDtypeStruct((1, _B), jnp.int32),
            jax.ShapeDtypeStruct((1, 1), jnp.float32),
        ],
        scratch_shapes=[
            pltpu.VMEM((1, _BM), jnp.float32),
            pltpu.VMEM((1, _BM), jnp.int32),
        ],
    )
    c_norm = jnp.sum(codebook * codebook, axis=1, keepdims=True)
    return call(z_e.T, -2.0 * codebook, c_norm)


@functools.cache
def _make_sc_gather():
    info = plsc.get_sparse_core_info()
    nc, ns = info.num_cores, info.num_subcores
    nw = nc * ns
    bpw = _B // nw
    nch = bpw // _IDX_CHUNK
    mesh = plsc.VectorSubcoreMesh(core_axis_name="c", subcore_axis_name="s")

    @functools.partial(
        pl.kernel,
        mesh=mesh,
        compiler_params=pltpu.CompilerParams(use_tc_tiling_on_sc=False),
        out_type=jax.ShapeDtypeStruct((_B, _D), jnp.float32),
        scratch_types=[
            pltpu.VMEM((nch, _IDX_CHUNK), jnp.int32),
            pltpu.VMEM((bpw, _D), jnp.float32),
            pltpu.SemaphoreType.DMA,
        ],
    )
    def gather(cb_hbm, idx_hbm, out_hbm, idx_v, rows_v, sem):
        wid = lax.axis_index("s") * nc + lax.axis_index("c")
        pltpu.sync_copy(idx_hbm.at[wid], idx_v)
        copies = []
        for j in range(nch):
            copies.append(pltpu.async_copy(
                cb_hbm.at[idx_v.at[j]],
                rows_v.at[pl.ds(j * _IDX_CHUNK, _IDX_CHUNK)],
                sem))
        for cp in copies:
            cp.wait()
        pltpu.sync_copy(rows_v, out_hbm.at[pl.ds(wid * bpw, bpw)])

    return gather, nw, nch


def kernel(z_e, codebook):
    z_e = z_e.astype(jnp.float32)
    idx2, commit2 = _tc_search(z_e, codebook)
    idx = idx2.reshape(_B)
    sc_gather, nw, nch = _make_sc_gather()
    z_q = sc_gather(codebook, idx.reshape(nw, nch, _IDX_CHUNK))
    return (z_q, idx, commit2[0, 0])


# speedup vs baseline: 1.0784x; 1.0784x over previous
import functools

import jax
import jax.numpy as jnp
from jax import lax
from jax.experimental import pallas as pl
from jax.experimental.pallas import tpu as pltpu
from jax.experimental.pallas import tpu_sc as plsc

_K = 8192
_D = 32
_B = 65536
_BETA = 0.25
_BM = 2048
_KC = 4096
_NJ = _K // _KC
_IDX_CHUNK = 128


def _tc_body(zt_ref, cb2_ref, cn_ref, idx_ref, commit_ref, bd_ref, bm_ref):
    j = pl.program_id(1)
    zt = zt_ref[...]
    cb2 = cb2_ref[pl.ds(j * _KC, _KC), :]
    prod = lax.dot_general(cb2, zt, (((1,), (0,)), ((), ())),
                           preferred_element_type=jnp.float32)
    dists = prod + cn_ref[pl.ds(j * _KC, _KC), :]
    d = jnp.min(dists, axis=0, keepdims=True)
    m = (jnp.argmin(dists, axis=0).astype(jnp.int32)
         .reshape(1, _BM) + j * _KC)

    @pl.when(j == 0)
    def _():
        bd_ref[...] = d
        bm_ref[...] = m

    @pl.when(j != 0)
    def _():
        prev_d = bd_ref[...]
        take = d < prev_d
        bd_ref[...] = jnp.where(take, d, prev_d)
        bm_ref[...] = jnp.where(take, m, bm_ref[...])

    @pl.when((pl.program_id(0) == 0) & (j == 0))
    def _():
        commit_ref[0, 0] = 0.0

    @pl.when(j == _NJ - 1)
    def _():
        idx_ref[...] = bm_ref[...]
        z = zt_ref[...]
        scale = _BETA / (_B * _D)
        commit_ref[0, 0] += (jnp.sum(bd_ref[...]) + jnp.sum(z * z)) * scale


def _tc_search(z_e, codebook, interpret=False):
    call = pl.pallas_call(
        _tc_body,
        interpret=interpret,
        grid=(_B // _BM, _NJ),
        in_specs=[
            pl.BlockSpec((_D, _BM), lambda i, j: (0, i)),
            pl.BlockSpec((_K, _D), lambda i, j: (0, 0)),
            pl.BlockSpec((_K, 1), lambda i, j: (0, 0)),
        ],
        out_specs=[
            pl.BlockSpec((1, _BM), lambda i, j: (0, i)),
            pl.BlockSpec(memory_space=pltpu.SMEM),
        ],
        out_shape=[
            jax.ShapeDtypeStruct((1, _B), jnp.int32),
            jax.ShapeDtypeStruct((1, 1), jnp.float32),
        ],
        scratch_shapes=[
            pltpu.VMEM((1, _BM), jnp.float32),
            pltpu.VMEM((1, _BM), jnp.int32),
        ],
    )
    c_norm = jnp.sum(codebook * codebook, axis=1, keepdims=True)
    return call(z_e.T, -2.0 * codebook, c_norm)


@functools.cache
def _make_sc_gather():
    info = plsc.get_sparse_core_info()
    nc, ns = info.num_cores, info.num_subcores
    nw = nc * ns
    bpw = _B // nw
    nch = bpw // _IDX_CHUNK
    mesh = plsc.VectorSubcoreMesh(core_axis_name="c", subcore_axis_name="s")

    @functools.partial(
        pl.kernel,
        mesh=mesh,
        compiler_params=pltpu.CompilerParams(use_tc_tiling_on_sc=False),
        out_type=jax.ShapeDtypeStruct((_B, _D), jnp.float32),
        scratch_types=[
            pltpu.VMEM((nch, _IDX_CHUNK), jnp.int32),
            pltpu.VMEM((bpw, _D), jnp.float32),
            pltpu.SemaphoreType.DMA,
        ],
    )
    def gather(cb_hbm, idx_hbm, out_hbm, idx_v, rows_v, sem):
        wid = lax.axis_index("s") * nc + lax.axis_index("c")
        pltpu.sync_copy(idx_hbm.at[wid], idx_v)
        copies = []
        for j in range(nch):
            copies.append(pltpu.async_copy(
                cb_hbm.at[idx_v.at[j]],
                rows_v.at[pl.ds(j * _IDX_CHUNK, _IDX_CHUNK)],
                sem))
        for cp in copies:
            cp.wait()
        pltpu.sync_copy(rows_v, out_hbm.at[pl.ds(wid * bpw, bpw)])

    return gather, nw, nch


def kernel(z_e, codebook):
    z_e = z_e.astype(jnp.float32)
    idx2, commit2 = _tc_search(z_e, codebook)
    idx = idx2.reshape(_B)
    sc_gather, nw, nch = _make_sc_gather()
    z_q = sc_gather(codebook, idx.reshape(nw, nch, _IDX_CHUNK))
    return (z_q, idx, commit2[0, 0])
